# baseline (speedup 1.0000x reference)
import jax
import jax.numpy as jnp
from jax.experimental import pallas as pl


def _leaky_body(x_ref, o_ref):
    x = x_ref[...]
    o_ref[...] = jnp.where(x > 0, x, 0.01 * x)


def _leaky(x):
    return pl.pallas_call(
        _leaky_body,
        out_shape=jax.ShapeDtypeStruct(x.shape, x.dtype),
    )(x)


def kernel(x_node_features, edge_index, edge_weight, W1, b1, W2, b2, W_out, b_out):
    src = edge_index[0]
    dst = edge_index[1]
    n = x_node_features.shape[0]

    def gcn_conv(x, W, b):
        h = x @ W
        msg = h[src] * edge_weight[:, None]
        agg = jax.ops.segment_sum(msg, dst, num_segments=n)
        return agg + b

    h1 = gcn_conv(x_node_features, W1, b1)
    h1 = _leaky(h1)
    h2 = gcn_conv(h1, W2, b2)
    out = h2 @ W_out + b_out
    return _leaky(out)


# speedup vs baseline: 2.5024x; 2.5024x over previous
import functools

import jax
import jax.numpy as jnp
from jax import lax
from jax.experimental import pallas as pl
from jax.experimental.pallas import tpu as pltpu
from jax.experimental.pallas import tpu_sc as plsc

N = 10000
D = 256
E = 160000
CH = 128
E_PAD = 163840
NCHUNK = E_PAD // CH
NSUB = 16
CPT = NCHUNK // NSUB
N_PAD = 10240
RPT = N_PAD // NSUB
BN = 400
NB = N // BN


def _leaky(x):
    return jnp.where(x >= 0, x, 0.01 * x)


def _mm1_body(x_ref, w_ref, o_ref):
    o_ref[0] = jnp.dot(x_ref[...], w_ref[...], preferred_element_type=jnp.float32)


def _mm1(x, W1):
    return pl.pallas_call(
        _mm1_body,
        grid=(2, NB),
        in_specs=[
            pl.BlockSpec((BN, D), lambda c, i: (i, 0)),
            pl.BlockSpec((D, 128), lambda c, i: (0, c)),
        ],
        out_specs=pl.BlockSpec((1, BN, 128), lambda c, i: (c, i, 0)),
        out_shape=jax.ShapeDtypeStruct((2, N, 128), jnp.float32),
    )(x, W1)


def _mm2_body(a_ref, b1_ref, w2_ref, o_ref):
    g0 = _leaky(a_ref[0] + b1_ref[0, :128])
    g1 = _leaky(a_ref[1] + b1_ref[0, 128:])
    o_ref[0] = (jnp.dot(g0, w2_ref[:128, :], preferred_element_type=jnp.float32)
                + jnp.dot(g1, w2_ref[128:, :], preferred_element_type=jnp.float32))


def _mm2(agg1, b1, W2):
    return pl.pallas_call(
        _mm2_body,
        grid=(2, NB),
        in_specs=[
            pl.BlockSpec((2, BN, 128), lambda c, i: (0, i, 0)),
            pl.BlockSpec((1, D), lambda c, i: (0, 0)),
            pl.BlockSpec((D, 128), lambda c, i: (0, c)),
        ],
        out_specs=pl.BlockSpec((1, BN, 128), lambda c, i: (c, i, 0)),
        out_shape=jax.ShapeDtypeStruct((2, N, 128), jnp.float32),
    )(agg1, b1.reshape(1, D), W2)


def _mm3_body(a_ref, b2_ref, wo_ref, bo_ref, o_ref):
    h0 = a_ref[0] + b2_ref[0, :128]
    h1 = a_ref[1] + b2_ref[0, 128:]
    t = (jnp.dot(h0, wo_ref[:128, :], preferred_element_type=jnp.float32)
         + jnp.dot(h1, wo_ref[128:, :], preferred_element_type=jnp.float32)
         + bo_ref[0])
    o_ref[...] = _leaky(t)


def _mm3(agg2, b2, W_out, b_out):
    return pl.pallas_call(
        _mm3_body,
        grid=(NB,),
        in_specs=[
            pl.BlockSpec((2, BN, 128), lambda i: (0, i, 0)),
            pl.BlockSpec((1, D), lambda i: (0, 0)),
            pl.BlockSpec((D, D), lambda i: (0, 0)),
            pl.BlockSpec((1, D), lambda i: (0, 0)),
        ],
        out_specs=pl.BlockSpec((BN, D), lambda i: (i, 0)),
        out_shape=jax.ShapeDtypeStruct((N, D), jnp.float32),
    )(agg2, b2.reshape(1, D), W_out, b_out.reshape(1, D))


def _sc_aggregate(h_split, src2d, dst2d, w2d, zeros_half):
    mesh = plsc.VectorSubcoreMesh(core_axis_name="c", subcore_axis_name="s")

    @functools.partial(
        pl.kernel,
        out_type=jax.ShapeDtypeStruct((2, N_PAD, 128), jnp.float32),
        mesh=mesh,
        scratch_types=[
            pltpu.VMEM((CPT, CH), jnp.int32),
            pltpu.VMEM((CPT, CH), jnp.int32),
            pltpu.VMEM((CPT, CH), jnp.float32),
            pltpu.VMEM((CH, 128), jnp.float32),
            pltpu.VMEM_SHARED((N_PAD, 128), jnp.float32),
            pltpu.SemaphoreType.DMA,
        ],
    )
    def agg(h_hbm, src_hbm, dst_hbm, w_hbm, z_hbm, out_hbm,
            src_v, dst_v, w_v, rows_v, acc, gsem):
        c = lax.axis_index("c")
        s = lax.axis_index("s")
        base = s * CPT
        pltpu.sync_copy(src_hbm.at[pl.ds(base, CPT)], src_v)
        pltpu.sync_copy(dst_hbm.at[pl.ds(base, CPT)], dst_v)
        pltpu.sync_copy(w_hbm.at[pl.ds(base, CPT)], w_v)
        rows0 = s * RPT
        pltpu.sync_copy(z_hbm.at[pl.ds(rows0, RPT)], acc.at[pl.ds(rows0, RPT)])
        plsc.subcore_barrier()

        def chunk_body(j, carry):
            pltpu.async_copy(h_hbm.at[c].at[src_v.at[j]], rows_v, gsem).wait()

            def group_body(g, carry2):
                wv16 = w_v[j, pl.ds(g * 16, 16)]
                for l in range(16):
                    wl = wv16[l]
                    e = g * 16 + l
                    for k in range(8):
                        sl = pl.ds(k * 16, 16)
                        rows_v[e, sl] = rows_v[e, sl] * wl
                return carry2

            lax.fori_loop(0, CH // 16, group_body, 0)
            pltpu.sync_copy(rows_v, acc.at[dst_v.at[j]], add=True)
            return carry

        lax.fori_loop(0, CPT, chunk_body, 0)
        plsc.subcore_barrier()
        pltpu.sync_copy(acc.at[pl.ds(rows0, RPT)],
                        out_hbm.at[c].at[pl.ds(rows0, RPT)])

    return agg(h_split, src2d, dst2d, w2d, zeros_half)


def kernel(x_node_features, edge_index, edge_weight, W1, b1, W2, b2, W_out, b_out):
    src = edge_index[0].astype(jnp.int32)
    dst = edge_index[1].astype(jnp.int32)
    w = edge_weight.astype(jnp.float32)
    pad = E_PAD - E
    src2d = jnp.concatenate([src, jnp.zeros((pad,), jnp.int32)]).reshape(NCHUNK, CH)
    dst2d = jnp.concatenate([dst, jnp.zeros((pad,), jnp.int32)]).reshape(NCHUNK, CH)
    w2d = jnp.concatenate([w, jnp.zeros((pad,), jnp.float32)]).reshape(NCHUNK, CH)
    zeros_half = jnp.zeros((N_PAD, 128), jnp.float32)

    h = _mm1(x_node_features, W1)
    agg1 = _sc_aggregate(h, src2d, dst2d, w2d, zeros_half)
    g2 = _mm2(agg1, b1, W2)
    agg2 = _sc_aggregate(g2, src2d, dst2d, w2d, zeros_half)
    return _mm3(agg2, b2, W_out, b_out)


# speedup vs baseline: 2.9449x; 1.1768x over previous
import functools

import jax
import jax.numpy as jnp
from jax import lax
from jax.experimental import pallas as pl
from jax.experimental.pallas import tpu as pltpu
from jax.experimental.pallas import tpu_sc as plsc

N = 10000
D = 256
E = 160000
CH = 128
E_PAD = 163840
NCHUNK = E_PAD // CH
NSUB = 16
CPT = NCHUNK // NSUB
PH_ROWS = 16
N_PAD = 10240
RPT = N_PAD // NSUB
BN = 400
NB = N // BN


def _leaky(x):
    return jnp.where(x >= 0, x, 0.01 * x)


def _mm1_body(x_ref, w_ref, o_ref):
    o_ref[0] = jnp.dot(x_ref[...], w_ref[...], preferred_element_type=jnp.float32)


def _mm1(x, W1):
    return pl.pallas_call(
        _mm1_body,
        grid=(2, NB),
        in_specs=[
            pl.BlockSpec((BN, D), lambda c, i: (i, 0)),
            pl.BlockSpec((D, 128), lambda c, i: (0, c)),
        ],
        out_specs=pl.BlockSpec((1, BN, 128), lambda c, i: (c, i, 0)),
        out_shape=jax.ShapeDtypeStruct((2, N, 128), jnp.float32),
    )(x, W1)


def _mm2_body(a_ref, b1_ref, w2_ref, o_ref):
    g0 = _leaky(a_ref[0] + b1_ref[0, :128])
    g1 = _leaky(a_ref[1] + b1_ref[0, 128:])
    o_ref[0] = (jnp.dot(g0, w2_ref[:128, :], preferred_element_type=jnp.float32)
                + jnp.dot(g1, w2_ref[128:, :], preferred_element_type=jnp.float32))


def _mm2(agg1, b1, W2):
    return pl.pallas_call(
        _mm2_body,
        grid=(2, NB),
        in_specs=[
            pl.BlockSpec((2, BN, 128), lambda c, i: (0, i, 0)),
            pl.BlockSpec((1, D), lambda c, i: (0, 0)),
            pl.BlockSpec((D, 128), lambda c, i: (0, c)),
        ],
        out_specs=pl.BlockSpec((1, BN, 128), lambda c, i: (c, i, 0)),
        out_shape=jax.ShapeDtypeStruct((2, N, 128), jnp.float32),
    )(agg1, b1.reshape(1, D), W2)


def _mm3_body(a_ref, b2_ref, wo_ref, bo_ref, o_ref):
    h0 = a_ref[0] + b2_ref[0, :128]
    h1 = a_ref[1] + b2_ref[0, 128:]
    t = (jnp.dot(h0, wo_ref[:128, :], preferred_element_type=jnp.float32)
         + jnp.dot(h1, wo_ref[128:, :], preferred_element_type=jnp.float32)
         + bo_ref[0])
    o_ref[...] = _leaky(t)


def _mm3(agg2, b2, W_out, b_out):
    return pl.pallas_call(
        _mm3_body,
        grid=(NB,),
        in_specs=[
            pl.BlockSpec((2, BN, 128), lambda i: (0, i, 0)),
            pl.BlockSpec((1, D), lambda i: (0, 0)),
            pl.BlockSpec((D, D), lambda i: (0, 0)),
            pl.BlockSpec((1, D), lambda i: (0, 0)),
        ],
        out_specs=pl.BlockSpec((BN, D), lambda i: (i, 0)),
        out_shape=jax.ShapeDtypeStruct((N, D), jnp.float32),
    )(agg2, b2.reshape(1, D), W_out, b_out.reshape(1, D))


def _sc_aggregate(h_split, src2d, dst2d, w2d, zeros_half):
    mesh = plsc.VectorSubcoreMesh(core_axis_name="c", subcore_axis_name="s")

    @functools.partial(
        pl.kernel,
        out_type=jax.ShapeDtypeStruct((2, N_PAD, 128), jnp.float32),
        mesh=mesh,
        scratch_types=[
            pltpu.VMEM((PH_ROWS, CH), jnp.int32),
            pltpu.VMEM((PH_ROWS, CH), jnp.int32),
            pltpu.VMEM((PH_ROWS, CH), jnp.float32),
            pltpu.VMEM((CH, 128), jnp.float32),
            pltpu.VMEM((CH, 128), jnp.float32),
            pltpu.VMEM_SHARED((N_PAD, 128), jnp.float32),
            pltpu.SemaphoreType.DMA,
            pltpu.SemaphoreType.DMA,
        ],
    )
    def agg(h_hbm, src_hbm, dst_hbm, w_hbm, z_hbm, out_hbm,
            src_v, dst_v, w_v, rows0_v, rows1_v, acc, gsem0, gsem1):
        c = lax.axis_index("c")
        s = lax.axis_index("s")
        rows0 = s * RPT
        pltpu.sync_copy(z_hbm.at[pl.ds(rows0, RPT)], acc.at[pl.ds(rows0, RPT)])
        plsc.subcore_barrier()

        hsrc = h_hbm.at[c]

        def scale_and_scatter(j, rows_v):
            def group_body(g, carry2):
                wv16 = w_v[j, pl.ds(g * 16, 16)]
                for l in range(16):
                    wl = wv16[l]
                    e = g * 16 + l
                    for k in range(8):
                        sl = pl.ds(k * 16, 16)
                        rows_v[e, sl] = rows_v[e, sl] * wl
                return carry2

            lax.fori_loop(0, CH // 16, group_body, 0)
            pltpu.sync_copy(rows_v, acc.at[dst_v.at[j]], add=True)

        def phase_body(p, carry):
            base = s * CPT + p * PH_ROWS
            pltpu.sync_copy(src_hbm.at[pl.ds(base, PH_ROWS)], src_v)
            pltpu.sync_copy(dst_hbm.at[pl.ds(base, PH_ROWS)], dst_v)
            pltpu.sync_copy(w_hbm.at[pl.ds(base, PH_ROWS)], w_v)

            pltpu.async_copy(hsrc.at[src_v.at[0]], rows0_v, gsem0)

            def pair_body(t, carry2):
                j0 = 2 * t
                pltpu.async_copy(hsrc.at[src_v.at[j0 + 1]], rows1_v, gsem1)
                pltpu.make_async_copy(hsrc.at[src_v.at[j0]], rows0_v, gsem0).wait()
                scale_and_scatter(j0, rows0_v)
                nxt = jnp.minimum(j0 + 2, PH_ROWS - 1)
                pltpu.async_copy(hsrc.at[src_v.at[nxt]], rows0_v, gsem0)
                pltpu.make_async_copy(hsrc.at[src_v.at[j0 + 1]], rows1_v, gsem1).wait()
                scale_and_scatter(j0 + 1, rows1_v)
                return carry2

            lax.fori_loop(0, PH_ROWS // 2, pair_body, 0)
            pltpu.make_async_copy(hsrc.at[src_v.at[PH_ROWS - 1]], rows0_v, gsem0).wait()
            return carry

        lax.fori_loop(0, CPT // PH_ROWS, phase_body, 0)
        plsc.subcore_barrier()
        pltpu.sync_copy(acc.at[pl.ds(rows0, RPT)],
                        out_hbm.at[c].at[pl.ds(rows0, RPT)])

    return agg(h_split, src2d, dst2d, w2d, zeros_half)


def kernel(x_node_features, edge_index, edge_weight, W1, b1, W2, b2, W_out, b_out):
    src = edge_index[0].astype(jnp.int32)
    dst = edge_index[1].astype(jnp.int32)
    w = edge_weight.astype(jnp.float32)
    pad = E_PAD - E
    src2d = jnp.concatenate([src, jnp.zeros((pad,), jnp.int32)]).reshape(NCHUNK, CH)
    dst2d = jnp.concatenate([dst, jnp.zeros((pad,), jnp.int32)]).reshape(NCHUNK, CH)
    w2d = jnp.concatenate([w, jnp.zeros((pad,), jnp.float32)]).reshape(NCHUNK, CH)
    zeros_half = jnp.zeros((N_PAD, 128), jnp.float32)

    h = _mm1(x_node_features, W1)
    agg1 = _sc_aggregate(h, src2d, dst2d, w2d, zeros_half)
    g2 = _mm2(agg1, b1, W2)
    agg2 = _sc_aggregate(g2, src2d, dst2d, w2d, zeros_half)
    return _mm3(agg2, b2, W_out, b_out)
